# baseline (speedup 1.0000x reference)
import functools

import jax
import jax.numpy as jnp
from jax import lax
from jax.experimental import pallas as pl
from jax.experimental.pallas import tpu as pltpu
from jax.experimental.pallas import tpu_sc as plsc

NC = 2
NS = 16
NW = NC * NS

IR = 128
CR = 4
NBUF = 2


def _build(n_rows, d_w, d_f):
  rows_per_w = n_rows // NW
  n_chunks = rows_per_w // CR
  assert n_chunks * CR == rows_per_w and n_chunks >= 4
  d_out = d_w + d_f
  n_tok = n_rows * IR
  ct = CR * IR
  mesh = plsc.VectorSubcoreMesh(
      core_axis_name="c", subcore_axis_name="s",
      num_cores=NC, num_subcores=NS)

  scratch = (
      [pltpu.VMEM((CR, IR), jnp.int32) for _ in range(NBUF)]
      + [pltpu.VMEM((CR, IR), jnp.int32) for _ in range(NBUF)]
      + [pltpu.VMEM((ct, d_w), jnp.float32) for _ in range(NBUF)]
      + [pltpu.VMEM((ct, d_f), jnp.float32) for _ in range(NBUF)]
      + [pltpu.SemaphoreType.DMA] * (4 * NBUF)
  )

  @functools.partial(
      pl.kernel,
      out_type=jax.ShapeDtypeStruct((n_tok, d_out), jnp.float32),
      mesh=mesh,
      scratch_types=scratch,
      compiler_params=pltpu.CompilerParams(use_tc_tiling_on_sc=False),
  )
  def body(x_hbm, y_hbm, wv_hbm, ft_hbm, out_hbm, *scr):
    xi = scr[0:NBUF]
    yi = scr[NBUF:2 * NBUF]
    wbuf = scr[2 * NBUF:3 * NBUF]
    fbuf = scr[3 * NBUF:4 * NBUF]
    sems = scr[4 * NBUF:]
    sem_idx = sems[0:NBUF]
    sem_gw = sems[NBUF:2 * NBUF]
    sem_gf = sems[2 * NBUF:3 * NBUF]
    sem_wr = sems[3 * NBUF:4 * NBUF]

    wid = lax.axis_index("s") * NC + lax.axis_index("c")
    row0 = wid * rows_per_w

    def idx_copies(g, b):
      r = row0 + g * CR
      return (pltpu.make_async_copy(x_hbm.at[pl.ds(r, CR)], xi[b], sem_idx[b]),
              pltpu.make_async_copy(y_hbm.at[pl.ds(r, CR)], yi[b], sem_idx[b]))

    def gather_copies(g, b):
      del g
      cs = []
      for j in range(CR):
        cs.append(pltpu.make_async_copy(
            wv_hbm.at[xi[b].at[j]], wbuf[b].at[pl.ds(j * IR, IR)], sem_gw[b]))
        cs.append(pltpu.make_async_copy(
            ft_hbm.at[yi[b].at[j]], fbuf[b].at[pl.ds(j * IR, IR)], sem_gf[b]))
      return cs

    def write_copies(g, b):
      t0 = (row0 + g * CR) * IR
      return (pltpu.make_async_copy(
                  wbuf[b], out_hbm.at[pl.ds(t0, ct), pl.ds(0, d_w)], sem_wr[b]),
              pltpu.make_async_copy(
                  fbuf[b], out_hbm.at[pl.ds(t0, ct), pl.ds(d_w, d_f)], sem_wr[b]))

    def start(copies):
      for c in copies:
        c.start()

    def wait(copies):
      for c in copies:
        c.wait()

    def run_chunk(g, b, prefetch, drain_prev):
      if drain_prev:
        wait(write_copies(g - NBUF, b))
      wait(idx_copies(g, b))
      gc = gather_copies(g, b)
      start(gc)
      wait(gc)
      start(write_copies(g, b))
      if prefetch:
        start(idx_copies(g + NBUF, b))

    for b in range(NBUF):
      start(idx_copies(b, b))
    for b in range(NBUF):
      run_chunk(b, b, prefetch=True, drain_prev=False)

    def loop_body(i, carry):
      g0 = NBUF + i * NBUF
      for b in range(NBUF):
        run_chunk(g0 + b, b, prefetch=True, drain_prev=True)
      return carry

    n_steady = (n_chunks - 2 * NBUF) // NBUF
    lax.fori_loop(0, n_steady, loop_body, 0)

    for b in range(NBUF):
      g = n_chunks - NBUF + b
      run_chunk(g, b, prefetch=False, drain_prev=True)
    for b in range(NBUF):
      wait(write_copies(n_chunks - NBUF + b, b))

  return body


def kernel(x, y, word_vectors, f_table):
  b, h = x.shape
  n_tok = b * h
  d_w = word_vectors.shape[1]
  d_f = f_table.shape[1]
  n_rows = n_tok // IR
  x2 = x.reshape(n_rows, IR).astype(jnp.int32)
  y2 = y.reshape(n_rows, IR).astype(jnp.int32)
  body = _build(n_rows, d_w, d_f)
  out = body(x2, y2, word_vectors, f_table)
  return out.reshape(b, h, d_w + d_f)


# speedup vs baseline: 3.6421x; 3.6421x over previous
import functools

import jax
import jax.numpy as jnp
from jax import lax
from jax.experimental import pallas as pl
from jax.experimental.pallas import tpu as pltpu
from jax.experimental.pallas import tpu_sc as plsc

NC = 2
NS = 16
NW = NC * NS

IR = 128
CR = 4
NBUF = 2


def _build(n_rows, d_w, d_f):
  rows_per_w = n_rows // NW
  n_chunks = rows_per_w // CR
  assert n_chunks * CR == rows_per_w and n_chunks >= 4
  d_out = d_w + d_f
  n_tok = n_rows * IR
  ct = CR * IR
  mesh = plsc.VectorSubcoreMesh(
      core_axis_name="c", subcore_axis_name="s",
      num_cores=NC, num_subcores=NS)

  scratch = (
      [pltpu.VMEM((CR, IR), jnp.int32) for _ in range(NBUF)]
      + [pltpu.VMEM((CR, IR), jnp.int32) for _ in range(NBUF)]
      + [pltpu.VMEM((ct, d_w), jnp.float32) for _ in range(NBUF)]
      + [pltpu.VMEM((ct, d_f), jnp.float32) for _ in range(NBUF)]
      + [pltpu.VMEM((4, d_f), jnp.float32)]
      + [pltpu.SemaphoreType.DMA] * (4 * NBUF)
  )

  @functools.partial(
      pl.kernel,
      out_type=jax.ShapeDtypeStruct((n_tok, d_out), jnp.float32),
      mesh=mesh,
      scratch_types=scratch,
      compiler_params=pltpu.CompilerParams(
          use_tc_tiling_on_sc=False, needs_layout_passes=False),
  )
  def body(x_hbm, y_hbm, wv_hbm, ft_hbm, out_hbm, *scr):
    xi = scr[0:NBUF]
    yi = scr[NBUF:2 * NBUF]
    wbuf = scr[2 * NBUF:3 * NBUF]
    fbuf = scr[3 * NBUF:4 * NBUF]
    fvm = scr[4 * NBUF]
    sems = scr[4 * NBUF + 1:]
    sem_idx = sems[0:NBUF]
    sem_gw = sems[NBUF:2 * NBUF]
    sem_gf = sems[2 * NBUF:3 * NBUF]
    sem_wr = sems[3 * NBUF:4 * NBUF]

    wid = lax.axis_index("s") * NC + lax.axis_index("c")
    row0 = wid * rows_per_w
    pltpu.sync_copy(ft_hbm, fvm)

    def idx_copies(g, b):
      r = row0 + g * CR
      return (pltpu.make_async_copy(x_hbm.at[pl.ds(r, CR)], xi[b], sem_idx[b]),
              pltpu.make_async_copy(y_hbm.at[pl.ds(r, CR)], yi[b], sem_idx[b]))

    def gather_copies(g, b):
      del g
      cs = []
      for j in range(CR):
        cs.append(pltpu.make_async_copy(
            wv_hbm.at[xi[b].at[j]], wbuf[b].at[pl.ds(j * IR, IR)], sem_gw[b]))
      return cs

    def write_copies(g, b):
      t0 = (row0 + g * CR) * IR
      return (pltpu.make_async_copy(
                  wbuf[b], out_hbm.at[pl.ds(t0, ct), pl.ds(0, d_w)], sem_wr[b]),
              pltpu.make_async_copy(
                  fbuf[b], out_hbm.at[pl.ds(t0, ct), pl.ds(d_w, d_f)], sem_wr[b]))

    def start(copies):
      for c in copies:
        c.start()

    def wait(copies):
      for c in copies:
        c.wait()

    LANES = 16
    iota = jnp.arange(LANES, dtype=jnp.int32)

    def expand_f(b):
      for j in range(CR):
        def tok(i, carry, j=j):
          k0 = i * LANES
          yv = yi[b][j, pl.ds(k0, LANES)]
          rows = j * IR + k0 + iota
          for c in range(d_f):
            col = jnp.full((LANES,), c, dtype=jnp.int32)
            vals = plsc.load_gather(fvm, [yv, col])
            plsc.store_scatter(fbuf[b], [rows, col], vals)
          return carry
        lax.fori_loop(0, IR // LANES, tok, 0, unroll=2)

    def run_chunk(g, b, prefetch, drain_prev):
      if drain_prev:
        wait(write_copies(g - NBUF, b))
      wait(idx_copies(g, b))
      gc = gather_copies(g, b)
      start(gc)
      expand_f(b)
      wait(gc)
      start(write_copies(g, b))
      if prefetch:
        start(idx_copies(g + NBUF, b))

    for b in range(NBUF):
      start(idx_copies(b, b))
    for b in range(NBUF):
      run_chunk(b, b, prefetch=True, drain_prev=False)

    def loop_body(i, carry):
      g0 = NBUF + i * NBUF
      for b in range(NBUF):
        run_chunk(g0 + b, b, prefetch=True, drain_prev=True)
      return carry

    n_steady = (n_chunks - 2 * NBUF) // NBUF
    lax.fori_loop(0, n_steady, loop_body, 0)

    for b in range(NBUF):
      g = n_chunks - NBUF + b
      run_chunk(g, b, prefetch=False, drain_prev=True)
    for b in range(NBUF):
      wait(write_copies(n_chunks - NBUF + b, b))

  return body


def kernel(x, y, word_vectors, f_table):
  b, h = x.shape
  n_tok = b * h
  d_w = word_vectors.shape[1]
  d_f = f_table.shape[1]
  n_rows = n_tok // IR
  x2 = x.reshape(n_rows, IR).astype(jnp.int32)
  y2 = y.reshape(n_rows, IR).astype(jnp.int32)
  body = _build(n_rows, d_w, d_f)
  out = body(x2, y2, word_vectors, f_table)
  return out.reshape(b, h, d_w + d_f)
